# Initial kernel scaffold; baseline (speedup 1.0000x reference)
import jax
import jax.numpy as jnp
from jax.experimental import pallas as pl


def kernel(points, W0, b0, gamma0, beta0, W1, b1, gamma1, beta1):
    raise NotImplementedError("write your pallas kernel here")


# speedup vs baseline: 8.6990x; 8.6990x over previous
import jax
import jax.numpy as jnp
from jax.experimental import pallas as pl
from jax.experimental.pallas import tpu as pltpu

_LEAKY = 0.2
_EPS = 1e-5
_NEG = -1.0e30

_R = 256
_RN = 1024


def _knn_kernel(xf_ref, xbt_ref, nbr_ref, *, n, k_top):
    xf = xf_ref[0]
    xbt = xbt_ref[0]
    inner = jax.lax.dot_general(
        xbt, xf, (((1,), (0,)), ((), ())),
        preferred_element_type=jnp.float32)
    xxf = jnp.sum(xf * xf, axis=0)
    xxb = jnp.sum(xbt * xbt, axis=1)
    d = 2.0 * inner - xxb[:, None] - xxf[None, :]
    col = jax.lax.broadcasted_iota(jnp.int32, d.shape, 1)
    for k in range(k_top):
        m = jnp.max(d, axis=1, keepdims=True)
        eq = d >= m
        j = jnp.min(jnp.where(eq, col, n), axis=1, keepdims=True)
        onehot_b = col == j
        onehot = onehot_b.astype(jnp.float32)
        nbr = jax.lax.dot_general(
            xf, onehot, (((1,), (1,)), ((), ())),
            preferred_element_type=jnp.float32)
        nbr_ref[0, k] = nbr
        d = jnp.where(onehot_b, _NEG, d)


def _stats0_kernel(c_ref, nbr_ref, a_ref, bm_ref, b0_ref, s1_ref, s2_ref):
    first = (pl.program_id(0) == 0) & (pl.program_id(1) == 0) & (pl.program_id(2) == 0)
    c = c_ref[0]
    nb = nbr_ref[0, 0]
    h0 = (jax.lax.dot_general(a_ref[...], c, (((1,), (0,)), ((), ())),
                              preferred_element_type=jnp.float32)
          + jax.lax.dot_general(bm_ref[...], nb, (((1,), (0,)), ((), ())),
                                preferred_element_type=jnp.float32)
          + b0_ref[...])
    s1 = jnp.sum(h0, axis=1, keepdims=True)
    s2 = jnp.sum(h0 * h0, axis=1, keepdims=True)
    s1_ref[...] = jnp.where(first, s1, s1_ref[...] + s1)
    s2_ref[...] = jnp.where(first, s2, s2_ref[...] + s2)


def _stats1_kernel(c_ref, nbr_ref, afs_ref, bfs_ref, c0_ref, w1_ref, b1_ref,
                   s1_ref, s2_ref):
    first = (pl.program_id(0) == 0) & (pl.program_id(1) == 0) & (pl.program_id(2) == 0)
    c = c_ref[0]
    nb = nbr_ref[0, 0]
    h0 = (jax.lax.dot_general(afs_ref[...], c, (((1,), (0,)), ((), ())),
                              preferred_element_type=jnp.float32)
          + jax.lax.dot_general(bfs_ref[...], nb, (((1,), (0,)), ((), ())),
                                preferred_element_type=jnp.float32)
          + c0_ref[...])
    g0 = jnp.where(h0 >= 0, h0, _LEAKY * h0)
    h1 = jax.lax.dot_general(w1_ref[...], g0, (((1,), (0,)), ((), ())),
                             preferred_element_type=jnp.float32) + b1_ref[...]
    s1 = jnp.sum(h1, axis=1, keepdims=True)
    s2 = jnp.sum(h1 * h1, axis=1, keepdims=True)
    s1_ref[...] = jnp.where(first, s1, s1_ref[...] + s1)
    s2_ref[...] = jnp.where(first, s2, s2_ref[...] + s2)


def _final_kernel(c_ref, nbr_ref, afs_ref, bfs_ref, c0_ref, w1s_ref, c1_ref,
                  out_ref):
    k = pl.program_id(2)
    c = c_ref[0]
    nb = nbr_ref[0, 0]
    h0 = (jax.lax.dot_general(afs_ref[...], c, (((1,), (0,)), ((), ())),
                              preferred_element_type=jnp.float32)
          + jax.lax.dot_general(bfs_ref[...], nb, (((1,), (0,)), ((), ())),
                                preferred_element_type=jnp.float32)
          + c0_ref[...])
    g0 = jnp.where(h0 >= 0, h0, _LEAKY * h0)
    h1 = jax.lax.dot_general(w1s_ref[...], g0, (((1,), (0,)), ((), ())),
                             preferred_element_type=jnp.float32) + c1_ref[...]
    y = jnp.where(h1 >= 0, h1, _LEAKY * h1)
    out_ref[0] = jnp.where(k == 0, y, jnp.maximum(out_ref[0], y))


def kernel(points, W0, b0, gamma0, beta0, W1, b1, gamma1, beta1):
    B, C, N = points.shape
    K = 20
    O0 = W0.shape[0]
    O1 = W1.shape[0]
    xt = jnp.transpose(points, (0, 2, 1))

    import functools
    knn = pl.pallas_call(
        functools.partial(_knn_kernel, n=N, k_top=K),
        grid=(B, N // _R),
        in_specs=[
            pl.BlockSpec((1, C, N), lambda b, nb: (b, 0, 0)),
            pl.BlockSpec((1, _R, C), lambda b, nb: (b, nb, 0)),
        ],
        out_specs=pl.BlockSpec((1, K, C, _R), lambda b, nb: (b, 0, 0, nb)),
        out_shape=jax.ShapeDtypeStruct((B, K, C, N), jnp.float32),
    )
    nbr = knn(points, xt)

    A = W0[:, :C] - W0[:, C:]
    Bm = W0[:, C:]
    b0c = b0[:, None]

    grid = (B, N // _RN, K)
    c_spec = pl.BlockSpec((1, C, _RN), lambda b, nb, k: (b, 0, nb))
    nbr_spec = pl.BlockSpec((1, 1, C, _RN), lambda b, nb, k: (b, k, 0, nb))
    w3_spec = pl.BlockSpec((O0, C), lambda b, nb, k: (0, 0))
    v_spec = pl.BlockSpec((O0, 1), lambda b, nb, k: (0, 0))
    s_spec = pl.BlockSpec((O0, 1), lambda b, nb, k: (0, 0))
    s_shape = jax.ShapeDtypeStruct((O0, 1), jnp.float32)

    s1, s2 = pl.pallas_call(
        _stats0_kernel,
        grid=grid,
        in_specs=[c_spec, nbr_spec, w3_spec, w3_spec, v_spec],
        out_specs=(s_spec, s_spec),
        out_shape=(s_shape, s_shape),
    )(points, nbr, A, Bm, b0c)

    M = B * N * K
    mean0 = s1[:, 0] / M
    var0 = s2[:, 0] / M - mean0 * mean0
    sc0 = gamma0 / jnp.sqrt(var0 + _EPS)
    Afs = A * sc0[:, None]
    Bfs = Bm * sc0[:, None]
    c0 = (b0 * sc0 + beta0 - mean0 * sc0)[:, None]

    w1_spec = pl.BlockSpec((O1, O0), lambda b, nb, k: (0, 0))
    t1, t2 = pl.pallas_call(
        _stats1_kernel,
        grid=grid,
        in_specs=[c_spec, nbr_spec, w3_spec, w3_spec, v_spec, w1_spec, v_spec],
        out_specs=(s_spec, s_spec),
        out_shape=(s_shape, s_shape),
    )(points, nbr, Afs, Bfs, c0, W1, b1[:, None])

    mean1 = t1[:, 0] / M
    var1 = t2[:, 0] / M - mean1 * mean1
    sc1 = gamma1 / jnp.sqrt(var1 + _EPS)
    W1s = W1 * sc1[:, None]
    c1 = (b1 * sc1 + beta1 - mean1 * sc1)[:, None]

    out = pl.pallas_call(
        _final_kernel,
        grid=grid,
        in_specs=[c_spec, nbr_spec, w3_spec, w3_spec, v_spec, w1_spec, v_spec],
        out_specs=pl.BlockSpec((1, O1, _RN), lambda b, nb, k: (b, 0, nb)),
        out_shape=jax.ShapeDtypeStruct((B, O1, N), jnp.float32),
    )(points, nbr, Afs, Bfs, c0, W1s, c1)
    return out


# speedup vs baseline: 8.9343x; 1.0270x over previous
import functools
import jax
import jax.numpy as jnp
from jax.experimental import pallas as pl

_LEAKY = 0.2
_EPS = 1e-5
_NEG = -1.0e30

_R = 256
_RN = 2048


def _knn_kernel(xf_ref, xt_ref, xbt_ref, nbr_ref, *, n, k_top):
    xf = xf_ref[0]
    xt = xt_ref[0]
    xbt = xbt_ref[0]
    inner = jax.lax.dot_general(
        xbt, xf, (((1,), (0,)), ((), ())),
        preferred_element_type=jnp.float32)
    xxf = jnp.sum(xf * xf, axis=0)
    xxb = jnp.sum(xbt * xbt, axis=1)
    d = 2.0 * inner - xxb[:, None] - xxf[None, :]
    col = jax.lax.broadcasted_iota(jnp.int32, d.shape, 1)
    for k in range(k_top):
        m = jnp.max(d, axis=1, keepdims=True)
        j = jnp.min(jnp.where(d >= m, col, n), axis=1, keepdims=True)
        onehot_b = col == j
        nbr = jax.lax.dot_general(
            onehot_b.astype(jnp.float32), xt, (((1,), (0,)), ((), ())),
            preferred_element_type=jnp.float32)
        nbr_ref[0, k] = nbr
        d = jnp.where(onehot_b, _NEG, d)


def _stats0_kernel(c_ref, nbr_ref, a_ref, bm_ref, b0_ref, s1_ref, s2_ref):
    first = (pl.program_id(0) == 0) & (pl.program_id(1) == 0) & (pl.program_id(2) == 0)
    c = c_ref[0]
    nb = nbr_ref[0, 0]
    h0 = (jax.lax.dot_general(c, a_ref[...], (((1,), (0,)), ((), ())),
                              preferred_element_type=jnp.float32)
          + jax.lax.dot_general(nb, bm_ref[...], (((1,), (0,)), ((), ())),
                                preferred_element_type=jnp.float32)
          + b0_ref[...])
    s1 = jnp.sum(h0, axis=0, keepdims=True)
    s2 = jnp.sum(h0 * h0, axis=0, keepdims=True)
    s1_ref[...] = jnp.where(first, s1, s1_ref[...] + s1)
    s2_ref[...] = jnp.where(first, s2, s2_ref[...] + s2)


def _stats1_kernel(c_ref, nbr_ref, afs_ref, bfs_ref, c0_ref, w1_ref, b1_ref,
                   s1_ref, s2_ref):
    first = (pl.program_id(0) == 0) & (pl.program_id(1) == 0) & (pl.program_id(2) == 0)
    c = c_ref[0]
    nb = nbr_ref[0, 0]
    h0 = (jax.lax.dot_general(c, afs_ref[...], (((1,), (0,)), ((), ())),
                              preferred_element_type=jnp.float32)
          + jax.lax.dot_general(nb, bfs_ref[...], (((1,), (0,)), ((), ())),
                                preferred_element_type=jnp.float32)
          + c0_ref[...])
    g0 = jnp.where(h0 >= 0, h0, _LEAKY * h0)
    h1 = jax.lax.dot_general(g0, w1_ref[...], (((1,), (0,)), ((), ())),
                             preferred_element_type=jnp.float32) + b1_ref[...]
    s1 = jnp.sum(h1, axis=0, keepdims=True)
    s2 = jnp.sum(h1 * h1, axis=0, keepdims=True)
    s1_ref[...] = jnp.where(first, s1, s1_ref[...] + s1)
    s2_ref[...] = jnp.where(first, s2, s2_ref[...] + s2)


def _final_kernel(c_ref, nbr_ref, afs_ref, bfs_ref, c0_ref, w1s_ref, c1_ref,
                  out_ref):
    k = pl.program_id(2)
    c = c_ref[0]
    nb = nbr_ref[0, 0]
    h0 = (jax.lax.dot_general(c, afs_ref[...], (((1,), (0,)), ((), ())),
                              preferred_element_type=jnp.float32)
          + jax.lax.dot_general(nb, bfs_ref[...], (((1,), (0,)), ((), ())),
                                preferred_element_type=jnp.float32)
          + c0_ref[...])
    g0 = jnp.where(h0 >= 0, h0, _LEAKY * h0)
    h1 = jax.lax.dot_general(g0, w1s_ref[...], (((1,), (0,)), ((), ())),
                             preferred_element_type=jnp.float32) + c1_ref[...]
    y = jnp.where(h1 >= 0, h1, _LEAKY * h1)
    out_ref[0] = jnp.where(k == 0, y, jnp.maximum(out_ref[0], y))


def kernel(points, W0, b0, gamma0, beta0, W1, b1, gamma1, beta1):
    B, C, N = points.shape
    K = 20
    O0 = W0.shape[0]
    O1 = W1.shape[0]
    xt = jnp.transpose(points, (0, 2, 1))

    knn = pl.pallas_call(
        functools.partial(_knn_kernel, n=N, k_top=K),
        grid=(B, N // _R),
        in_specs=[
            pl.BlockSpec((1, C, N), lambda b, nb: (b, 0, 0)),
            pl.BlockSpec((1, N, C), lambda b, nb: (b, 0, 0)),
            pl.BlockSpec((1, _R, C), lambda b, nb: (b, nb, 0)),
        ],
        out_specs=pl.BlockSpec((1, K, _R, C), lambda b, nb: (b, 0, nb, 0)),
        out_shape=jax.ShapeDtypeStruct((B, K, N, C), jnp.float32),
    )
    nbr = knn(points, xt, xt)

    A = (W0[:, :C] - W0[:, C:]).T
    Bm = W0[:, C:].T
    b0r = b0[None, :]

    grid = (B, N // _RN, K)
    c_spec = pl.BlockSpec((1, _RN, C), lambda b, nb, k: (b, nb, 0))
    nbr_spec = pl.BlockSpec((1, 1, _RN, C), lambda b, nb, k: (b, k, nb, 0))
    w3_spec = pl.BlockSpec((C, O0), lambda b, nb, k: (0, 0))
    v_spec = pl.BlockSpec((1, O0), lambda b, nb, k: (0, 0))
    s_shape = jax.ShapeDtypeStruct((1, O0), jnp.float32)

    s1, s2 = pl.pallas_call(
        _stats0_kernel,
        grid=grid,
        in_specs=[c_spec, nbr_spec, w3_spec, w3_spec, v_spec],
        out_specs=(v_spec, v_spec),
        out_shape=(s_shape, s_shape),
    )(xt, nbr, A, Bm, b0r)

    M = B * N * K
    mean0 = s1[0] / M
    var0 = s2[0] / M - mean0 * mean0
    sc0 = gamma0 / jnp.sqrt(var0 + _EPS)
    Afs = A * sc0[None, :]
    Bfs = Bm * sc0[None, :]
    c0 = (b0 * sc0 + beta0 - mean0 * sc0)[None, :]

    w1_spec = pl.BlockSpec((O0, O1), lambda b, nb, k: (0, 0))
    t1, t2 = pl.pallas_call(
        _stats1_kernel,
        grid=grid,
        in_specs=[c_spec, nbr_spec, w3_spec, w3_spec, v_spec, w1_spec, v_spec],
        out_specs=(v_spec, v_spec),
        out_shape=(s_shape, s_shape),
    )(xt, nbr, Afs, Bfs, c0, W1.T, b1[None, :])

    mean1 = t1[0] / M
    var1 = t2[0] / M - mean1 * mean1
    sc1 = gamma1 / jnp.sqrt(var1 + _EPS)
    W1s = W1.T * sc1[None, :]
    c1 = (b1 * sc1 + beta1 - mean1 * sc1)[None, :]

    out = pl.pallas_call(
        _final_kernel,
        grid=grid,
        in_specs=[c_spec, nbr_spec, w3_spec, w3_spec, v_spec, w1_spec, v_spec],
        out_specs=pl.BlockSpec((1, _RN, O1), lambda b, nb, k: (b, nb, 0)),
        out_shape=jax.ShapeDtypeStruct((B, N, O1), jnp.float32),
    )(xt, nbr, Afs, Bfs, c0, W1s, c1)
    return jnp.transpose(out, (0, 2, 1))


# speedup vs baseline: 9.7730x; 1.0939x over previous
import functools
import jax
import jax.numpy as jnp
from jax.experimental import pallas as pl

_LEAKY = 0.2
_EPS = 1e-5
_NEG = -1.0e30

_R = 256
_RN = 2048


def _knn_kernel(xf_ref, xt_ref, xbt_ref, nbr_ref, *, n, k_top):
    xf = xf_ref[0]
    xt = xt_ref[0]
    xbt = xbt_ref[0]
    inner = jax.lax.dot_general(
        xbt, xf, (((1,), (0,)), ((), ())),
        preferred_element_type=jnp.float32)
    xxf = jnp.sum(xf * xf, axis=0)
    xxb = jnp.sum(xbt * xbt, axis=1)
    d = 2.0 * inner - xxb[:, None] - xxf[None, :]
    col = jax.lax.broadcasted_iota(jnp.int32, d.shape, 1)
    for k in range(k_top):
        j = jnp.argmax(d, axis=1).astype(jnp.int32)[:, None]
        onehot_b = col == j
        nbr = jax.lax.dot_general(
            onehot_b.astype(jnp.float32), xt, (((1,), (0,)), ((), ())),
            preferred_element_type=jnp.float32)
        nbr_ref[0, k] = nbr
        d = jnp.where(onehot_b, _NEG, d)


def _stats0_kernel(c_ref, nbr_ref, a_ref, bm_ref, b0_ref, s1_ref, s2_ref):
    first = (pl.program_id(0) == 0) & (pl.program_id(1) == 0) & (pl.program_id(2) == 0)
    c = c_ref[0]
    nb = nbr_ref[0, 0]
    h0 = (jax.lax.dot_general(c, a_ref[...], (((1,), (0,)), ((), ())),
                              preferred_element_type=jnp.float32)
          + jax.lax.dot_general(nb, bm_ref[...], (((1,), (0,)), ((), ())),
                                preferred_element_type=jnp.float32)
          + b0_ref[...])
    s1 = jnp.sum(h0, axis=0, keepdims=True)
    s2 = jnp.sum(h0 * h0, axis=0, keepdims=True)
    s1_ref[...] = jnp.where(first, s1, s1_ref[...] + s1)
    s2_ref[...] = jnp.where(first, s2, s2_ref[...] + s2)


def _stats1_kernel(c_ref, nbr_ref, afs_ref, bfs_ref, c0_ref, w1_ref, b1_ref,
                   s1_ref, s2_ref):
    first = (pl.program_id(0) == 0) & (pl.program_id(1) == 0) & (pl.program_id(2) == 0)
    c = c_ref[0]
    nb = nbr_ref[0, 0]
    h0 = (jax.lax.dot_general(c, afs_ref[...], (((1,), (0,)), ((), ())),
                              preferred_element_type=jnp.float32)
          + jax.lax.dot_general(nb, bfs_ref[...], (((1,), (0,)), ((), ())),
                                preferred_element_type=jnp.float32)
          + c0_ref[...])
    g0 = jnp.where(h0 >= 0, h0, _LEAKY * h0)
    h1 = jax.lax.dot_general(g0, w1_ref[...], (((1,), (0,)), ((), ())),
                             preferred_element_type=jnp.float32) + b1_ref[...]
    s1 = jnp.sum(h1, axis=0, keepdims=True)
    s2 = jnp.sum(h1 * h1, axis=0, keepdims=True)
    s1_ref[...] = jnp.where(first, s1, s1_ref[...] + s1)
    s2_ref[...] = jnp.where(first, s2, s2_ref[...] + s2)


def _final_kernel(c_ref, nbr_ref, afs_ref, bfs_ref, c0_ref, w1s_ref, c1_ref,
                  out_ref):
    k = pl.program_id(2)
    c = c_ref[0]
    nb = nbr_ref[0, 0]
    h0 = (jax.lax.dot_general(c, afs_ref[...], (((1,), (0,)), ((), ())),
                              preferred_element_type=jnp.float32)
          + jax.lax.dot_general(nb, bfs_ref[...], (((1,), (0,)), ((), ())),
                                preferred_element_type=jnp.float32)
          + c0_ref[...])
    g0 = jnp.where(h0 >= 0, h0, _LEAKY * h0)
    h1 = jax.lax.dot_general(g0, w1s_ref[...], (((1,), (0,)), ((), ())),
                             preferred_element_type=jnp.float32) + c1_ref[...]
    y = jnp.where(h1 >= 0, h1, _LEAKY * h1)
    out_ref[0] = jnp.where(k == 0, y, jnp.maximum(out_ref[0], y))


def kernel(points, W0, b0, gamma0, beta0, W1, b1, gamma1, beta1):
    B, C, N = points.shape
    K = 20
    O0 = W0.shape[0]
    O1 = W1.shape[0]
    xt = jnp.transpose(points, (0, 2, 1))

    knn = pl.pallas_call(
        functools.partial(_knn_kernel, n=N, k_top=K),
        grid=(B, N // _R),
        in_specs=[
            pl.BlockSpec((1, C, N), lambda b, nb: (b, 0, 0)),
            pl.BlockSpec((1, N, C), lambda b, nb: (b, 0, 0)),
            pl.BlockSpec((1, _R, C), lambda b, nb: (b, nb, 0)),
        ],
        out_specs=pl.BlockSpec((1, K, _R, C), lambda b, nb: (b, 0, nb, 0)),
        out_shape=jax.ShapeDtypeStruct((B, K, N, C), jnp.float32),
    )
    nbr = knn(points, xt, xt)

    A = (W0[:, :C] - W0[:, C:]).T
    Bm = W0[:, C:].T
    b0r = b0[None, :]

    grid = (B, N // _RN, K)
    c_spec = pl.BlockSpec((1, _RN, C), lambda b, nb, k: (b, nb, 0))
    nbr_spec = pl.BlockSpec((1, 1, _RN, C), lambda b, nb, k: (b, k, nb, 0))
    w3_spec = pl.BlockSpec((C, O0), lambda b, nb, k: (0, 0))
    v_spec = pl.BlockSpec((1, O0), lambda b, nb, k: (0, 0))
    s_shape = jax.ShapeDtypeStruct((1, O0), jnp.float32)

    s1, s2 = pl.pallas_call(
        _stats0_kernel,
        grid=grid,
        in_specs=[c_spec, nbr_spec, w3_spec, w3_spec, v_spec],
        out_specs=(v_spec, v_spec),
        out_shape=(s_shape, s_shape),
    )(xt, nbr, A, Bm, b0r)

    M = B * N * K
    mean0 = s1[0] / M
    var0 = s2[0] / M - mean0 * mean0
    sc0 = gamma0 / jnp.sqrt(var0 + _EPS)
    Afs = A * sc0[None, :]
    Bfs = Bm * sc0[None, :]
    c0 = (b0 * sc0 + beta0 - mean0 * sc0)[None, :]

    w1_spec = pl.BlockSpec((O0, O1), lambda b, nb, k: (0, 0))
    t1, t2 = pl.pallas_call(
        _stats1_kernel,
        grid=grid,
        in_specs=[c_spec, nbr_spec, w3_spec, w3_spec, v_spec, w1_spec, v_spec],
        out_specs=(v_spec, v_spec),
        out_shape=(s_shape, s_shape),
    )(xt, nbr, Afs, Bfs, c0, W1.T, b1[None, :])

    mean1 = t1[0] / M
    var1 = t2[0] / M - mean1 * mean1
    sc1 = gamma1 / jnp.sqrt(var1 + _EPS)
    W1s = W1.T * sc1[None, :]
    c1 = (b1 * sc1 + beta1 - mean1 * sc1)[None, :]

    out = pl.pallas_call(
        _final_kernel,
        grid=grid,
        in_specs=[c_spec, nbr_spec, w3_spec, w3_spec, v_spec, w1_spec, v_spec],
        out_specs=pl.BlockSpec((1, _RN, O1), lambda b, nb, k: (b, nb, 0)),
        out_shape=jax.ShapeDtypeStruct((B, N, O1), jnp.float32),
    )(xt, nbr, Afs, Bfs, c0, W1s, c1)
    return jnp.transpose(out, (0, 2, 1))


# speedup vs baseline: 11.9959x; 1.2275x over previous
import functools
import jax
import jax.numpy as jnp
from jax import lax
from jax.experimental import pallas as pl
from jax.experimental.pallas import tpu as pltpu
from jax.experimental.pallas import tpu_sc as plsc

_LEAKY = 0.2
_EPS = 1e-5
_NEG = -1.0e30

_R = 256
_RN = 2048


def _knn_kernel(xf_ref, xbt_ref, idx_ref, *, n, k_top):
    xf = xf_ref[0]
    xbt = xbt_ref[0]
    inner = jax.lax.dot_general(
        xbt, xf, (((1,), (0,)), ((), ())),
        preferred_element_type=jnp.float32)
    xxf = jnp.sum(xf * xf, axis=0)
    xxb = jnp.sum(xbt * xbt, axis=1)
    d = 2.0 * inner - xxb[:, None] - xxf[None, :]
    col = jax.lax.broadcasted_iota(jnp.int32, d.shape, 1)
    for k in range(k_top):
        j = jnp.argmax(d, axis=1).astype(jnp.int32)[:, None]
        idx_ref[0, :, k] = j[:, 0]
        d = jnp.where(col == j, _NEG, d)


def _sc_gather_kernel(idx_hbm, xt_hbm, out_hbm, idx_v, tab_v, chan_v, *,
                      tasks_per_worker, n, k_top, n_ch):
    wid = lax.axis_index("s") * 2 + lax.axis_index("c")

    def task(t_local, carry):
        t = wid * tasks_per_worker + t_local
        b = t // k_top
        pltpu.sync_copy(xt_hbm.at[b], tab_v)
        pltpu.sync_copy(idx_hbm.at[t], idx_v)

        def chunk(i, carry2):
            rows = idx_v[pl.ds(i * 16, 16)]
            off = rows * n_ch
            for c in range(n_ch):
                chan_v[pl.ds(c * n + i * 16, 16)] = plsc.load_gather(
                    tab_v, [off + c])
            return carry2

        lax.fori_loop(0, n // 16, chunk, 0)
        pltpu.sync_copy(chan_v, out_hbm.at[t])
        return carry

    lax.fori_loop(0, tasks_per_worker, task, 0)


def _stats0_kernel(c_ref, nbr_ref, a_ref, bm_ref, b0_ref, s1_ref, s2_ref):
    first = (pl.program_id(0) == 0) & (pl.program_id(1) == 0) & (pl.program_id(2) == 0)
    c = c_ref[0]
    nb = nbr_ref[0]
    h0 = (jax.lax.dot_general(a_ref[...], c, (((1,), (0,)), ((), ())),
                              preferred_element_type=jnp.float32)
          + jax.lax.dot_general(bm_ref[...], nb, (((1,), (0,)), ((), ())),
                                preferred_element_type=jnp.float32)
          + b0_ref[...])
    s1 = jnp.sum(h0, axis=1, keepdims=True)
    s2 = jnp.sum(h0 * h0, axis=1, keepdims=True)
    s1_ref[...] = jnp.where(first, s1, s1_ref[...] + s1)
    s2_ref[...] = jnp.where(first, s2, s2_ref[...] + s2)


def _stats1_kernel(c_ref, nbr_ref, afs_ref, bfs_ref, c0_ref, w1_ref, b1_ref,
                   s1_ref, s2_ref):
    first = (pl.program_id(0) == 0) & (pl.program_id(1) == 0) & (pl.program_id(2) == 0)
    c = c_ref[0]
    nb = nbr_ref[0]
    h0 = (jax.lax.dot_general(afs_ref[...], c, (((1,), (0,)), ((), ())),
                              preferred_element_type=jnp.float32)
          + jax.lax.dot_general(bfs_ref[...], nb, (((1,), (0,)), ((), ())),
                                preferred_element_type=jnp.float32)
          + c0_ref[...])
    g0 = jnp.where(h0 >= 0, h0, _LEAKY * h0)
    h1 = jax.lax.dot_general(w1_ref[...], g0, (((1,), (0,)), ((), ())),
                             preferred_element_type=jnp.float32) + b1_ref[...]
    s1 = jnp.sum(h1, axis=1, keepdims=True)
    s2 = jnp.sum(h1 * h1, axis=1, keepdims=True)
    s1_ref[...] = jnp.where(first, s1, s1_ref[...] + s1)
    s2_ref[...] = jnp.where(first, s2, s2_ref[...] + s2)


def _final_kernel(c_ref, nbr_ref, afs_ref, bfs_ref, c0_ref, w1s_ref, c1_ref,
                  out_ref):
    k = pl.program_id(2)
    c = c_ref[0]
    nb = nbr_ref[0]
    h0 = (jax.lax.dot_general(afs_ref[...], c, (((1,), (0,)), ((), ())),
                              preferred_element_type=jnp.float32)
          + jax.lax.dot_general(bfs_ref[...], nb, (((1,), (0,)), ((), ())),
                                preferred_element_type=jnp.float32)
          + c0_ref[...])
    g0 = jnp.where(h0 >= 0, h0, _LEAKY * h0)
    h1 = jax.lax.dot_general(w1s_ref[...], g0, (((1,), (0,)), ((), ())),
                             preferred_element_type=jnp.float32) + c1_ref[...]
    y = jnp.where(h1 >= 0, h1, _LEAKY * h1)
    out_ref[0] = jnp.where(k == 0, y, jnp.maximum(out_ref[0], y))


def kernel(points, W0, b0, gamma0, beta0, W1, b1, gamma1, beta1):
    B, C, N = points.shape
    K = 20
    O0 = W0.shape[0]
    O1 = W1.shape[0]
    xt = jnp.transpose(points, (0, 2, 1))

    knn = pl.pallas_call(
        functools.partial(_knn_kernel, n=N, k_top=K),
        grid=(B, N // _R),
        in_specs=[
            pl.BlockSpec((1, C, N), lambda b, nb: (b, 0, 0)),
            pl.BlockSpec((1, _R, C), lambda b, nb: (b, nb, 0)),
        ],
        out_specs=pl.BlockSpec((1, _R, K), lambda b, nb: (b, nb, 0)),
        out_shape=jax.ShapeDtypeStruct((B, N, K), jnp.int32),
    )
    idx = knn(points, xt)

    idx_t = jnp.transpose(idx, (0, 2, 1)).reshape(B * K, N)
    xt_flat = xt.reshape(B, N * C)
    n_workers = 32
    tasks_per_worker = (B * K) // n_workers
    mesh = plsc.VectorSubcoreMesh(core_axis_name="c", subcore_axis_name="s")
    sc_gather = pl.kernel(
        functools.partial(_sc_gather_kernel, tasks_per_worker=tasks_per_worker,
                          n=N, k_top=K, n_ch=C),
        mesh=mesh,
        compiler_params=pltpu.CompilerParams(needs_layout_passes=False),
        out_type=jax.ShapeDtypeStruct((B * K, C * N), jnp.float32),
        scratch_types=[
            pltpu.VMEM((N,), jnp.int32),
            pltpu.VMEM((N * C,), jnp.float32),
            pltpu.VMEM((C * N,), jnp.float32),
        ],
    )
    nbr = sc_gather(idx_t, xt_flat).reshape(B * K, C, N)

    A = W0[:, :C] - W0[:, C:]
    Bm = W0[:, C:]
    b0c = b0[:, None]

    grid = (B, N // _RN, K)
    c_spec = pl.BlockSpec((1, C, _RN), lambda b, nb, k: (b, 0, nb))
    nbr_spec = pl.BlockSpec((1, C, _RN), lambda b, nb, k: (b * K + k, 0, nb))
    w3_spec = pl.BlockSpec((O0, C), lambda b, nb, k: (0, 0))
    v_spec = pl.BlockSpec((O0, 1), lambda b, nb, k: (0, 0))
    s_shape = jax.ShapeDtypeStruct((O0, 1), jnp.float32)

    s1, s2 = pl.pallas_call(
        _stats0_kernel,
        grid=grid,
        in_specs=[c_spec, nbr_spec, w3_spec, w3_spec, v_spec],
        out_specs=(v_spec, v_spec),
        out_shape=(s_shape, s_shape),
    )(points, nbr, A, Bm, b0c)

    M = B * N * K
    mean0 = s1[:, 0] / M
    var0 = s2[:, 0] / M - mean0 * mean0
    sc0 = gamma0 / jnp.sqrt(var0 + _EPS)
    Afs = A * sc0[:, None]
    Bfs = Bm * sc0[:, None]
    c0 = (b0 * sc0 + beta0 - mean0 * sc0)[:, None]

    w1_spec = pl.BlockSpec((O1, O0), lambda b, nb, k: (0, 0))
    t1, t2 = pl.pallas_call(
        _stats1_kernel,
        grid=grid,
        in_specs=[c_spec, nbr_spec, w3_spec, w3_spec, v_spec, w1_spec, v_spec],
        out_specs=(v_spec, v_spec),
        out_shape=(s_shape, s_shape),
    )(points, nbr, Afs, Bfs, c0, W1, b1[:, None])

    mean1 = t1[:, 0] / M
    var1 = t2[:, 0] / M - mean1 * mean1
    sc1 = gamma1 / jnp.sqrt(var1 + _EPS)
    W1s = W1 * sc1[:, None]
    c1 = (b1 * sc1 + beta1 - mean1 * sc1)[:, None]

    out = pl.pallas_call(
        _final_kernel,
        grid=grid,
        in_specs=[c_spec, nbr_spec, w3_spec, w3_spec, v_spec, w1_spec, v_spec],
        out_specs=pl.BlockSpec((1, O1, _RN), lambda b, nb, k: (b, 0, nb)),
        out_shape=jax.ShapeDtypeStruct((B, O1, N), jnp.float32),
    )(points, nbr, Afs, Bfs, c0, W1s, c1)
    return out
